# baseline (speedup 1.0000x reference)
import jax
import jax.numpy as jnp
from jax.experimental import pallas as pl
from jax.experimental.pallas import tpu as pltpu

_NV = 10242


def _copy_kernel(x_ref, o_ref, sem):
    cp = pltpu.make_async_copy(x_ref.at[pl.ds(0, _NV)], o_ref, sem)
    cp.start()
    cp.wait()


def kernel(x):
    out_shape = jax.ShapeDtypeStruct((_NV,) + x.shape[1:], x.dtype)
    return pl.pallas_call(
        _copy_kernel,
        out_shape=out_shape,
        in_specs=[pl.BlockSpec(memory_space=pl.ANY)],
        out_specs=pl.BlockSpec(memory_space=pl.ANY),
        scratch_shapes=[pltpu.SemaphoreType.DMA],
    )(x)


# speedup vs baseline: 48.0392x; 48.0392x over previous
import jax
import jax.numpy as jnp
from jax.experimental import pallas as pl
from jax.experimental.pallas import tpu as pltpu

_NV = 10242
_R = 1536


def _copy_kernel(x_ref, o_ref):
    o_ref[...] = x_ref[...]


def kernel(x):
    out_shape = jax.ShapeDtypeStruct((_NV,) + x.shape[1:], x.dtype)
    grid = (pl.cdiv(_NV, _R),)
    return pl.pallas_call(
        _copy_kernel,
        out_shape=out_shape,
        grid=grid,
        in_specs=[pl.BlockSpec((_R, 16, 128), lambda i: (i, 0, 0))],
        out_specs=pl.BlockSpec((_R, 16, 128), lambda i: (i, 0, 0)),
    )(x)
